# baseline (speedup 1.0000x reference)
import functools

import jax
import jax.numpy as jnp
from jax import lax
from jax.experimental import pallas as pl
from jax.experimental.pallas import tpu as pltpu
from jax.experimental.pallas import tpu_sc as plsc

_B, _N, _D, _H, _DH = 8, 577, 768, 12, 64
_K = 57
_KP = 64
_C1_24 = 0.0416666679


def _norm_body(a1row_ref, a0_ref, q_ref, rr_ref, acc_ref):
    h = pl.program_id(0)
    blk = a0_ref[0, :, 0, :, :]

    @pl.when(h == 0)
    def _():
        acc_ref[...] = blk

    @pl.when(h > 0)
    def _():
        acc_ref[...] = acc_ref[...] + blk

    @pl.when(h == _H - 1)
    def _():
        ii = lax.broadcasted_iota(jnp.int32, (_N, _B, _N), 0)
        jj = lax.broadcasted_iota(jnp.int32, (_N, _B, _N), 2)
        t = acc_ref[...] * jnp.float32(_C1_24)
        t = t + jnp.where(ii == jj, jnp.float32(0.5), jnp.float32(0.0))
        s = jnp.sum(t, axis=2, keepdims=True)
        q_ref[...] = (t / s).astype(jnp.bfloat16)
        ar = a1row_ref[...]
        t1 = ar[:, 0, :]
        for hh in range(1, _H):
            t1 = t1 + ar[:, hh, :]
        t1 = t1 * jnp.float32(_C1_24)
        j1 = lax.broadcasted_iota(jnp.int32, (_B, _N), 1)
        t1 = t1 + jnp.where(j1 == 0, jnp.float32(0.5), jnp.float32(0.0))
        s1 = jnp.sum(t1, axis=1, keepdims=True)
        rr_ref[...] = (t1 / s1).astype(jnp.bfloat16)


def _cls_dot_body(rr_ref, qt_ref, cls_ref):
    b = pl.program_id(0)
    bi = lax.broadcasted_iota(jnp.int32, (_B, 1), 0)
    rsel = jnp.where(bi == b, rr_ref[...], jnp.bfloat16(0.0))
    rb = jnp.sum(rsel, axis=0, keepdims=True)
    cls_ref[0] = lax.dot_general(
        rb, qt_ref[0], (((1,), (0,)), ((), ())),
        preferred_element_type=jnp.float32)


def _cls_rollout(ah, a1row):
    aht = jnp.transpose(ah, (0, 3, 2, 1, 4))
    q, rr = pl.pallas_call(
        _norm_body,
        grid=(_H,),
        in_specs=[
            pl.BlockSpec((_B, _H, _N), lambda h: (0, 0, 0)),
            pl.BlockSpec((1, _N, 1, _B, _N), lambda h: (0, 0, h, 0, 0)),
        ],
        out_specs=[
            pl.BlockSpec((_N, _B, _N), lambda h: (0, 0, 0)),
            pl.BlockSpec((_B, _N), lambda h: (0, 0)),
        ],
        out_shape=[
            jax.ShapeDtypeStruct((_N, _B, _N), jnp.bfloat16),
            jax.ShapeDtypeStruct((_B, _N), jnp.bfloat16),
        ],
        scratch_shapes=[pltpu.VMEM((_N, _B, _N), jnp.float32)],
    )(a1row, aht[:1])
    qt = jnp.transpose(q, (1, 0, 2))
    return pl.pallas_call(
        _cls_dot_body,
        grid=(_B,),
        in_specs=[
            pl.BlockSpec((_B, _N), lambda b: (0, 0)),
            pl.BlockSpec((1, _N, _N), lambda b: (b, 0, 0)),
        ],
        out_specs=pl.BlockSpec((1, 1, _N), lambda b: (b, 0, 0)),
        out_shape=jax.ShapeDtypeStruct((_B, 1, _N), jnp.float32),
    )(rr, qt)


def _topk_sc(cls576):
    ncols = 576
    nch = ncols // 16
    int_min = jnp.int32(-2147483648)

    mesh = plsc.VectorSubcoreMesh(core_axis_name="c", subcore_axis_name="s")

    @functools.partial(
        pl.kernel,
        mesh=mesh,
        out_type=jax.ShapeDtypeStruct((_B, _KP), jnp.int32),
        scratch_types=[
            pltpu.VMEM((ncols,), jnp.float32),
            pltpu.VMEM((ncols,), jnp.int32),
            pltpu.VMEM((_KP,), jnp.int32),
        ],
    )
    def topk_kernel(cls_hbm, idx_hbm, row_v, key_v, out_v):
        wid = lax.axis_index("s") * 2 + lax.axis_index("c")

        @pl.when(wid < _B)
        def _():
            pltpu.sync_copy(cls_hbm.at[wid], row_v)
            lane = lax.broadcasted_iota(jnp.int32, (16,), 0)
            for c in range(nch):
                v = row_v[pl.ds(c * 16, 16)]
                k = lax.bitcast_convert_type(v, jnp.int32)
                k = jnp.where(k < 0, k ^ jnp.int32(0x7FFFFFFF), k)
                key_v[pl.ds(c * 16, 16)] = k

            def _lane_all_max(v):
                for sh in (8, 4, 2, 1):
                    v = jnp.maximum(
                        v, v.at[lane ^ sh].get(mode="promise_in_bounds"))
                return v

            def _lane_all_min(v):
                for sh in (8, 4, 2, 1):
                    v = jnp.minimum(
                        v, v.at[lane ^ sh].get(mode="promise_in_bounds"))
                return v

            zero16 = jnp.zeros((16,), jnp.int32)

            def step(t, carry):
                supp0, supp1, o0, o1, o2, o3 = carry
                best = jnp.full((16,), int_min, jnp.int32)
                besti = zero16
                for c in range(nch):
                    kv = key_v[pl.ds(c * 16, 16)]
                    if c < 32:
                        sup = (supp0 >> c) & 1
                    else:
                        sup = (supp1 >> (c - 32)) & 1
                    kv = jnp.where(sup == 1, jnp.full((16,), int_min,
                                                      jnp.int32), kv)
                    take = kv > best
                    best = jnp.where(take, kv, best)
                    besti = jnp.where(take, jnp.full((16,), c, jnp.int32),
                                      besti)
                m = _lane_all_max(best)
                gidx = besti * 16 + lane
                cand = jnp.where(best == m, gidx, jnp.int32(2147483647))
                mi = _lane_all_min(cand)
                ch = mi >> 4
                ln = mi & 15
                hit = lane == ln
                supp0 = jnp.where(hit & (ch < 32),
                                  supp0 | (jnp.int32(1) << ch), supp0)
                supp1 = jnp.where(hit & (ch >= 32),
                                  supp1 | (jnp.int32(1) << (ch - 32)), supp1)
                val = mi + 1
                tv = jnp.full((16,), t, jnp.int32)
                o0 = jnp.where(lane == tv, val, o0)
                o1 = jnp.where(lane + 16 == tv, val, o1)
                o2 = jnp.where(lane + 32 == tv, val, o2)
                o3 = jnp.where(lane + 48 == tv, val, o3)
                return supp0, supp1, o0, o1, o2, o3

            init = (zero16, zero16, zero16, zero16, zero16, zero16)
            _, _, o0, o1, o2, o3 = lax.fori_loop(0, _K, step, init)
            o3 = jnp.where(lane >= _K - 48, jnp.full((16,), 1, jnp.int32),
                           o3)
            out_v[pl.ds(0, 16)] = o0
            out_v[pl.ds(16, 16)] = o1
            out_v[pl.ds(32, 16)] = o2
            out_v[pl.ds(48, 16)] = o3
            pltpu.sync_copy(out_v, idx_hbm.at[wid])

    return topk_kernel(cls576)


def _ln_rows(xr, g, b):
    mu = jnp.mean(xr, axis=1, keepdims=True)
    d = xr - mu
    var = jnp.mean(d * d, axis=1, keepdims=True)
    return d / jnp.sqrt(var + jnp.float32(1e-5)) * g + b


def _kv_body(x_ref, g_ref, b_ref, wk_ref, bk_ref, wv_ref, bv_ref,
             k_ref, v_ref):
    hh = _ln_rows(x_ref[0], g_ref[...], b_ref[...]).astype(jnp.bfloat16)
    kk = lax.dot_general(hh, wk_ref[...], (((1,), (1,)), ((), ())),
                         preferred_element_type=jnp.float32)
    vv = lax.dot_general(hh, wv_ref[...], (((1,), (1,)), ((), ())),
                         preferred_element_type=jnp.float32)
    k_ref[0] = (kk + bk_ref[...]).astype(jnp.bfloat16)
    v_ref[0] = (vv + bv_ref[...]).astype(jnp.bfloat16)


def _ln_kv(x, ln1_g, ln1_b, wk_b, bk, wv_b, bv):
    wspec = pl.BlockSpec((_D, _D), lambda b: (0, 0))
    bspec = pl.BlockSpec((1, _D), lambda b: (0, 0))
    return pl.pallas_call(
        _kv_body,
        grid=(_B,),
        in_specs=[
            pl.BlockSpec((1, _N, _D), lambda b: (b, 0, 0)),
            bspec, bspec, wspec, bspec, wspec, bspec,
        ],
        out_specs=[
            pl.BlockSpec((1, _N, _D), lambda b: (b, 0, 0)),
            pl.BlockSpec((1, _N, _D), lambda b: (b, 0, 0)),
        ],
        out_shape=[
            jax.ShapeDtypeStruct((_B, _N, _D), jnp.bfloat16),
            jax.ShapeDtypeStruct((_B, _N, _D), jnp.bfloat16),
        ],
    )(x, ln1_g.reshape(1, _D), ln1_b.reshape(1, _D), wk_b, bk.reshape(1, _D),
      wv_b, bv.reshape(1, _D))


def _attn_body(idx_ref, x_ref, g_ref, b_ref, wq_ref, bq_ref, wo_ref, bo_ref,
               k_ref, v_ref, aw_ref, glca_ref, xs_ref, ctx_ref):
    b = pl.program_id(0)
    for t in range(_KP):
        iv = idx_ref[b, t]
        xs_ref[pl.ds(t, 1), :] = x_ref[0, pl.ds(iv, 1), :]
    hh = _ln_rows(xs_ref[...], g_ref[...], b_ref[...]).astype(jnp.bfloat16)
    q = lax.dot_general(hh, wq_ref[...], (((1,), (1,)), ((), ())),
                        preferred_element_type=jnp.float32) + bq_ref[...]
    q = q.astype(jnp.bfloat16)
    for head in range(_H):
        qh = q[:, head * _DH:(head + 1) * _DH]
        kh = k_ref[0][:, head * _DH:(head + 1) * _DH]
        vh = v_ref[0][:, head * _DH:(head + 1) * _DH]
        sc = lax.dot_general(qh, kh, (((1,), (1,)), ((), ())),
                             preferred_element_type=jnp.float32)
        sc = sc * jnp.float32(0.125)
        m = jnp.max(sc, axis=1, keepdims=True)
        e = jnp.exp(sc - m)
        w = e / jnp.sum(e, axis=1, keepdims=True)
        aw_ref[0, head] = w[0:_K, :]
        ch = lax.dot_general(w.astype(jnp.bfloat16), vh,
                             (((1,), (0,)), ((), ())),
                             preferred_element_type=jnp.float32)
        ctx_ref[:, head * _DH:(head + 1) * _DH] = ch
    gl = lax.dot_general(ctx_ref[...].astype(jnp.bfloat16), wo_ref[...],
                         (((1,), (1,)), ((), ())),
                         preferred_element_type=jnp.float32)
    glca_ref[0] = gl + bo_ref[...]


def _attention(idx, x, ln1_g, ln1_b, wq_b, bq, wo_b, bo, kk, vv):
    wspec = pl.BlockSpec((_D, _D), lambda b: (0, 0))
    bspec = pl.BlockSpec((1, _D), lambda b: (0, 0))
    xspec = pl.BlockSpec((1, _N, _D), lambda b: (b, 0, 0))
    return pl.pallas_call(
        _attn_body,
        grid=(_B,),
        in_specs=[
            pl.BlockSpec(memory_space=pltpu.SMEM),
            xspec, bspec, bspec, wspec, bspec, wspec, bspec, xspec, xspec,
        ],
        out_specs=[
            pl.BlockSpec((1, _H, _K, _N), lambda b: (b, 0, 0, 0)),
            pl.BlockSpec((1, _KP, _D), lambda b: (b, 0, 0)),
        ],
        out_shape=[
            jax.ShapeDtypeStruct((_B, _H, _K, _N), jnp.float32),
            jax.ShapeDtypeStruct((_B, _KP, _D), jnp.float32),
        ],
        scratch_shapes=[
            pltpu.VMEM((_KP, _D), jnp.float32),
            pltpu.VMEM((_KP, _D), jnp.float32),
        ],
    )(idx, x, ln1_g.reshape(1, _D), ln1_b.reshape(1, _D), wq_b,
      bq.reshape(1, _D), wo_b, bo.reshape(1, _D), kk, vv)


def _ffn_body(idx_ref, x_ref, glca_ref, g_ref, b_ref, w1_ref, b1_ref,
              w2_ref, b2_ref, out_ref):
    b = pl.program_id(0)
    out_ref[0] = x_ref[0]
    for t in range(_K):
        iv = idx_ref[b, t]
        out_ref[0, pl.ds(iv, 1), :] = (out_ref[0, pl.ds(iv, 1), :]
                                       + glca_ref[0, pl.ds(t, 1), :])
    x2 = out_ref[0]
    h2 = _ln_rows(x2, g_ref[...], b_ref[...]).astype(jnp.bfloat16)
    pre = lax.dot_general(h2, w1_ref[...], (((1,), (1,)), ((), ())),
                          preferred_element_type=jnp.float32) + b1_ref[...]
    act = (pre * jnp.float32(0.5)
           * (jnp.float32(1.0) + lax.erf(pre * jnp.float32(0.7071067811865476))))
    ff = lax.dot_general(act.astype(jnp.bfloat16), w2_ref[...],
                         (((1,), (1,)), ((), ())),
                         preferred_element_type=jnp.float32) + b2_ref[...]
    out_ref[0] = x2 + ff


def _ffn_scatter(idx, x, glca, ln2_g, ln2_b, w1_b, b1, w2_b, b2):
    return pl.pallas_call(
        _ffn_body,
        grid=(_B,),
        in_specs=[
            pl.BlockSpec(memory_space=pltpu.SMEM),
            pl.BlockSpec((1, _N, _D), lambda b: (b, 0, 0)),
            pl.BlockSpec((1, _KP, _D), lambda b: (b, 0, 0)),
            pl.BlockSpec((1, _D), lambda b: (0, 0)),
            pl.BlockSpec((1, _D), lambda b: (0, 0)),
            pl.BlockSpec((3072, _D), lambda b: (0, 0)),
            pl.BlockSpec((1, 3072), lambda b: (0, 0)),
            pl.BlockSpec((_D, 3072), lambda b: (0, 0)),
            pl.BlockSpec((1, _D), lambda b: (0, 0)),
        ],
        out_specs=pl.BlockSpec((1, _N, _D), lambda b: (b, 0, 0)),
        out_shape=jax.ShapeDtypeStruct((_B, _N, _D), jnp.float32),
    )(idx, x, glca, ln2_g.reshape(1, _D), ln2_b.reshape(1, _D), w1_b,
      b1.reshape(1, 3072), w2_b, b2.reshape(1, _D))


def kernel(x, attention_history, ln1_g, ln1_b, Wq, bq, Wk, bk, Wv, bv,
           Wo, bo, ln2_g, ln2_b, W1, b1, W2, b2):
    a1row = attention_history[1, :, :, 0, :]
    cls = _cls_rollout(attention_history, a1row)
    idx = _topk_sc(cls[:, 0, 1:])

    kk, vv = _ln_kv(x, ln1_g, ln1_b, Wk.astype(jnp.bfloat16), bk,
                    Wv.astype(jnp.bfloat16), bv)
    aw, glca = _attention(idx, x, ln1_g, ln1_b, Wq.astype(jnp.bfloat16), bq,
                          Wo.astype(jnp.bfloat16), bo, kk, vv)
    out = _ffn_scatter(idx, x, glca, ln2_g, ln2_b,
                       W1.astype(jnp.bfloat16), b1,
                       W2.astype(jnp.bfloat16), b2)
    return (out, aw)


# speedup vs baseline: 4.6042x; 4.6042x over previous
import functools

import jax
import jax.numpy as jnp
from jax import lax
from jax.experimental import pallas as pl
from jax.experimental.pallas import tpu as pltpu
from jax.experimental.pallas import tpu_sc as plsc

_B, _N, _D, _H, _DH = 8, 577, 768, 12, 64
_K = 57
_KP = 64
_C1_24 = 0.0416666679


def _norm_body(a1row_ref, a0_ref, q_ref, rr_ref, acc_ref):
    h = pl.program_id(0)
    blk = a0_ref[0, :, 0, :, :]

    @pl.when(h == 0)
    def _():
        acc_ref[...] = blk

    @pl.when(h > 0)
    def _():
        acc_ref[...] = acc_ref[...] + blk

    @pl.when(h == _H - 1)
    def _():
        ii = lax.broadcasted_iota(jnp.int32, (_N, _B, _N), 0)
        jj = lax.broadcasted_iota(jnp.int32, (_N, _B, _N), 2)
        t = acc_ref[...] * jnp.float32(_C1_24)
        t = t + jnp.where(ii == jj, jnp.float32(0.5), jnp.float32(0.0))
        s = jnp.sum(t, axis=2, keepdims=True)
        q_ref[...] = (t / s).astype(jnp.bfloat16)
        ar = a1row_ref[...]
        t1 = ar[:, 0, :]
        for hh in range(1, _H):
            t1 = t1 + ar[:, hh, :]
        t1 = t1 * jnp.float32(_C1_24)
        j1 = lax.broadcasted_iota(jnp.int32, (_B, _N), 1)
        t1 = t1 + jnp.where(j1 == 0, jnp.float32(0.5), jnp.float32(0.0))
        s1 = jnp.sum(t1, axis=1, keepdims=True)
        rr_ref[...] = (t1 / s1).astype(jnp.bfloat16)


def _cls_dot_body(rr_ref, qt_ref, cls_ref):
    b = pl.program_id(0)
    bi = lax.broadcasted_iota(jnp.int32, (_B, 1), 0)
    rsel = jnp.where(bi == b, rr_ref[...], jnp.bfloat16(0.0))
    rb = jnp.sum(rsel, axis=0, keepdims=True)
    cls_ref[0] = lax.dot_general(
        rb, qt_ref[0], (((1,), (0,)), ((), ())),
        preferred_element_type=jnp.float32)


def _cls_rollout(ah, a1row):
    aht = jnp.transpose(ah[0:1], (0, 3, 2, 1, 4))
    q, rr = pl.pallas_call(
        _norm_body,
        grid=(_H,),
        in_specs=[
            pl.BlockSpec((_B, _H, _N), lambda h: (0, 0, 0)),
            pl.BlockSpec((1, _N, 1, _B, _N), lambda h: (0, 0, h, 0, 0)),
        ],
        out_specs=[
            pl.BlockSpec((_N, _B, _N), lambda h: (0, 0, 0)),
            pl.BlockSpec((_B, _N), lambda h: (0, 0)),
        ],
        out_shape=[
            jax.ShapeDtypeStruct((_N, _B, _N), jnp.bfloat16),
            jax.ShapeDtypeStruct((_B, _N), jnp.bfloat16),
        ],
        scratch_shapes=[pltpu.VMEM((_N, _B, _N), jnp.float32)],
    )(a1row, aht[:1])
    qt = jnp.transpose(q, (1, 0, 2))
    return pl.pallas_call(
        _cls_dot_body,
        grid=(_B,),
        in_specs=[
            pl.BlockSpec((_B, _N), lambda b: (0, 0)),
            pl.BlockSpec((1, _N, _N), lambda b: (b, 0, 0)),
        ],
        out_specs=pl.BlockSpec((1, 1, _N), lambda b: (b, 0, 0)),
        out_shape=jax.ShapeDtypeStruct((_B, 1, _N), jnp.float32),
    )(rr, qt)


def _topk_sc(cls576):
    ncols = 576
    nch = ncols // 16
    int_min = jnp.int32(-2147483648)

    mesh = plsc.VectorSubcoreMesh(core_axis_name="c", subcore_axis_name="s")

    @functools.partial(
        pl.kernel,
        mesh=mesh,
        out_type=jax.ShapeDtypeStruct((_B, _KP), jnp.int32),
        scratch_types=[
            pltpu.VMEM((ncols,), jnp.float32),
            pltpu.VMEM((ncols,), jnp.int32),
            pltpu.VMEM((_KP,), jnp.int32),
        ],
    )
    def topk_kernel(cls_hbm, idx_hbm, row_v, key_v, out_v):
        wid = lax.axis_index("s") * 2 + lax.axis_index("c")

        @pl.when(wid < _B)
        def _():
            pltpu.sync_copy(cls_hbm.at[wid], row_v)
            lane = lax.broadcasted_iota(jnp.int32, (16,), 0)
            for c in range(nch):
                v = row_v[pl.ds(c * 16, 16)]
                k = lax.bitcast_convert_type(v, jnp.int32)
                k = jnp.where(k < 0, k ^ jnp.int32(0x7FFFFFFF), k)
                key_v[pl.ds(c * 16, 16)] = k

            def _lane_all_max(v):
                for sh in (8, 4, 2, 1):
                    v = jnp.maximum(
                        v, v.at[lane ^ sh].get(mode="promise_in_bounds"))
                return v

            def _lane_all_min(v):
                for sh in (8, 4, 2, 1):
                    v = jnp.minimum(
                        v, v.at[lane ^ sh].get(mode="promise_in_bounds"))
                return v

            zero16 = jnp.zeros((16,), jnp.int32)

            def step(t, carry):
                supp0, supp1, o0, o1, o2, o3 = carry
                best = jnp.full((16,), int_min, jnp.int32)
                besti = zero16
                for c in range(nch):
                    kv = key_v[pl.ds(c * 16, 16)]
                    if c < 32:
                        sup = (supp0 >> c) & 1
                    else:
                        sup = (supp1 >> (c - 32)) & 1
                    kv = jnp.where(sup == 1, jnp.full((16,), int_min,
                                                      jnp.int32), kv)
                    take = kv > best
                    best = jnp.where(take, kv, best)
                    besti = jnp.where(take, jnp.full((16,), c, jnp.int32),
                                      besti)
                m = _lane_all_max(best)
                gidx = besti * 16 + lane
                cand = jnp.where(best == m, gidx, jnp.int32(2147483647))
                mi = _lane_all_min(cand)
                ch = mi >> 4
                ln = mi & 15
                hit = lane == ln
                supp0 = jnp.where(hit & (ch < 32),
                                  supp0 | (jnp.int32(1) << ch), supp0)
                supp1 = jnp.where(hit & (ch >= 32),
                                  supp1 | (jnp.int32(1) << (ch - 32)), supp1)
                val = mi + 1
                tv = jnp.full((16,), t, jnp.int32)
                o0 = jnp.where(lane == tv, val, o0)
                o1 = jnp.where(lane + 16 == tv, val, o1)
                o2 = jnp.where(lane + 32 == tv, val, o2)
                o3 = jnp.where(lane + 48 == tv, val, o3)
                return supp0, supp1, o0, o1, o2, o3

            init = (zero16, zero16, zero16, zero16, zero16, zero16)
            _, _, o0, o1, o2, o3 = lax.fori_loop(0, _K, step, init)
            o3 = jnp.where(lane >= _K - 48, jnp.full((16,), 1, jnp.int32),
                           o3)
            out_v[pl.ds(0, 16)] = o0
            out_v[pl.ds(16, 16)] = o1
            out_v[pl.ds(32, 16)] = o2
            out_v[pl.ds(48, 16)] = o3
            pltpu.sync_copy(out_v, idx_hbm.at[wid])

    return topk_kernel(cls576)


def _ln_rows(xr, g, b):
    mu = jnp.mean(xr, axis=1, keepdims=True)
    d = xr - mu
    var = jnp.mean(d * d, axis=1, keepdims=True)
    return d / jnp.sqrt(var + jnp.float32(1e-5)) * g + b


def _kv_body(x_ref, g_ref, b_ref, wk_ref, bk_ref, wv_ref, bv_ref,
             k_ref, v_ref):
    hh = _ln_rows(x_ref[0], g_ref[...], b_ref[...]).astype(jnp.bfloat16)
    kk = lax.dot_general(hh, wk_ref[...], (((1,), (1,)), ((), ())),
                         preferred_element_type=jnp.float32)
    vv = lax.dot_general(hh, wv_ref[...], (((1,), (1,)), ((), ())),
                         preferred_element_type=jnp.float32)
    k_ref[0] = (kk + bk_ref[...]).astype(jnp.bfloat16)
    v_ref[0] = (vv + bv_ref[...]).astype(jnp.bfloat16)


def _ln_kv(x, ln1_g, ln1_b, wk_b, bk, wv_b, bv):
    wspec = pl.BlockSpec((_D, _D), lambda b: (0, 0))
    bspec = pl.BlockSpec((1, _D), lambda b: (0, 0))
    return pl.pallas_call(
        _kv_body,
        grid=(_B,),
        in_specs=[
            pl.BlockSpec((1, _N, _D), lambda b: (b, 0, 0)),
            bspec, bspec, wspec, bspec, wspec, bspec,
        ],
        out_specs=[
            pl.BlockSpec((1, _N, _D), lambda b: (b, 0, 0)),
            pl.BlockSpec((1, _N, _D), lambda b: (b, 0, 0)),
        ],
        out_shape=[
            jax.ShapeDtypeStruct((_B, _N, _D), jnp.bfloat16),
            jax.ShapeDtypeStruct((_B, _N, _D), jnp.bfloat16),
        ],
    )(x, ln1_g.reshape(1, _D), ln1_b.reshape(1, _D), wk_b, bk.reshape(1, _D),
      wv_b, bv.reshape(1, _D))


def _attn_body(idx_ref, x_ref, g_ref, b_ref, wq_ref, bq_ref, wo_ref, bo_ref,
               k_ref, v_ref, aw_ref, glca_ref, xs_ref, ctx_ref):
    b = pl.program_id(0)
    for t in range(_KP):
        iv = idx_ref[b, t]
        xs_ref[pl.ds(t, 1), :] = x_ref[0, pl.ds(iv, 1), :]
    hh = _ln_rows(xs_ref[...], g_ref[...], b_ref[...]).astype(jnp.bfloat16)
    q = lax.dot_general(hh, wq_ref[...], (((1,), (1,)), ((), ())),
                        preferred_element_type=jnp.float32) + bq_ref[...]
    q = q.astype(jnp.bfloat16)
    for head in range(_H):
        qh = q[:, head * _DH:(head + 1) * _DH]
        kh = k_ref[0][:, head * _DH:(head + 1) * _DH]
        vh = v_ref[0][:, head * _DH:(head + 1) * _DH]
        sc = lax.dot_general(qh, kh, (((1,), (1,)), ((), ())),
                             preferred_element_type=jnp.float32)
        sc = sc * jnp.float32(0.125)
        m = jnp.max(sc, axis=1, keepdims=True)
        e = jnp.exp(sc - m)
        w = e / jnp.sum(e, axis=1, keepdims=True)
        aw_ref[0, head] = w[0:_K, :]
        ch = lax.dot_general(w.astype(jnp.bfloat16), vh,
                             (((1,), (0,)), ((), ())),
                             preferred_element_type=jnp.float32)
        ctx_ref[:, head * _DH:(head + 1) * _DH] = ch
    gl = lax.dot_general(ctx_ref[...].astype(jnp.bfloat16), wo_ref[...],
                         (((1,), (1,)), ((), ())),
                         preferred_element_type=jnp.float32)
    glca_ref[0] = gl + bo_ref[...]


def _attention(idx, x, ln1_g, ln1_b, wq_b, bq, wo_b, bo, kk, vv):
    wspec = pl.BlockSpec((_D, _D), lambda b: (0, 0))
    bspec = pl.BlockSpec((1, _D), lambda b: (0, 0))
    xspec = pl.BlockSpec((1, _N, _D), lambda b: (b, 0, 0))
    return pl.pallas_call(
        _attn_body,
        grid=(_B,),
        in_specs=[
            pl.BlockSpec(memory_space=pltpu.SMEM),
            xspec, bspec, bspec, wspec, bspec, wspec, bspec, xspec, xspec,
        ],
        out_specs=[
            pl.BlockSpec((1, _H, _K, _N), lambda b: (b, 0, 0, 0)),
            pl.BlockSpec((1, _KP, _D), lambda b: (b, 0, 0)),
        ],
        out_shape=[
            jax.ShapeDtypeStruct((_B, _H, _K, _N), jnp.float32),
            jax.ShapeDtypeStruct((_B, _KP, _D), jnp.float32),
        ],
        scratch_shapes=[
            pltpu.VMEM((_KP, _D), jnp.float32),
            pltpu.VMEM((_KP, _D), jnp.float32),
        ],
    )(idx, x, ln1_g.reshape(1, _D), ln1_b.reshape(1, _D), wq_b,
      bq.reshape(1, _D), wo_b, bo.reshape(1, _D), kk, vv)


def _ffn_body(idx_ref, x_ref, glca_ref, g_ref, b_ref, w1_ref, b1_ref,
              w2_ref, b2_ref, out_ref):
    b = pl.program_id(0)
    out_ref[0] = x_ref[0]
    for t in range(_K):
        iv = idx_ref[b, t]
        out_ref[0, pl.ds(iv, 1), :] = (out_ref[0, pl.ds(iv, 1), :]
                                       + glca_ref[0, pl.ds(t, 1), :])
    x2 = out_ref[0]
    h2 = _ln_rows(x2, g_ref[...], b_ref[...]).astype(jnp.bfloat16)
    pre = lax.dot_general(h2, w1_ref[...], (((1,), (1,)), ((), ())),
                          preferred_element_type=jnp.float32) + b1_ref[...]
    act = (pre * jnp.float32(0.5)
           * (jnp.float32(1.0) + lax.erf(pre * jnp.float32(0.7071067811865476))))
    ff = lax.dot_general(act.astype(jnp.bfloat16), w2_ref[...],
                         (((1,), (1,)), ((), ())),
                         preferred_element_type=jnp.float32) + b2_ref[...]
    out_ref[0] = x2 + ff


def _ffn_scatter(idx, x, glca, ln2_g, ln2_b, w1_b, b1, w2_b, b2):
    return pl.pallas_call(
        _ffn_body,
        grid=(_B,),
        in_specs=[
            pl.BlockSpec(memory_space=pltpu.SMEM),
            pl.BlockSpec((1, _N, _D), lambda b: (b, 0, 0)),
            pl.BlockSpec((1, _KP, _D), lambda b: (b, 0, 0)),
            pl.BlockSpec((1, _D), lambda b: (0, 0)),
            pl.BlockSpec((1, _D), lambda b: (0, 0)),
            pl.BlockSpec((3072, _D), lambda b: (0, 0)),
            pl.BlockSpec((1, 3072), lambda b: (0, 0)),
            pl.BlockSpec((_D, 3072), lambda b: (0, 0)),
            pl.BlockSpec((1, _D), lambda b: (0, 0)),
        ],
        out_specs=pl.BlockSpec((1, _N, _D), lambda b: (b, 0, 0)),
        out_shape=jax.ShapeDtypeStruct((_B, _N, _D), jnp.float32),
    )(idx, x, glca, ln2_g.reshape(1, _D), ln2_b.reshape(1, _D), w1_b,
      b1.reshape(1, 3072), w2_b, b2.reshape(1, _D))


def kernel(x, attention_history, ln1_g, ln1_b, Wq, bq, Wk, bk, Wv, bv,
           Wo, bo, ln2_g, ln2_b, W1, b1, W2, b2):
    a1row = attention_history[1, :, :, 0, :]
    cls = _cls_rollout(attention_history, a1row)
    idx = _topk_sc(cls[:, 0, 1:])

    kk, vv = _ln_kv(x, ln1_g, ln1_b, Wk.astype(jnp.bfloat16), bk,
                    Wv.astype(jnp.bfloat16), bv)
    aw, glca = _attention(idx, x, ln1_g, ln1_b, Wq.astype(jnp.bfloat16), bq,
                          Wo.astype(jnp.bfloat16), bo, kk, vv)
    out = _ffn_scatter(idx, x, glca, ln2_g, ln2_b,
                       W1.astype(jnp.bfloat16), b1,
                       W2.astype(jnp.bfloat16), b2)
    return (out, aw)
